# baseline (speedup 1.0000x reference)
import functools

import jax
import jax.numpy as jnp
from jax.experimental import pallas as pl
from jax.experimental.pallas import tpu as pltpu

NUM_VARS = 128
K = 5
NSLOTS = 3
NH = 8


def _conv_body(xc_ref, w_ref, b_ref, out_hbm, xs_ref, sem):
    i = pl.program_id(0)
    nb = pl.num_programs(0)
    tt = xc_ref.shape[1]
    ht = tt // NH
    slot = jax.lax.rem(i, NSLOTS)
    n = NUM_VARS

    def _chunk_copy(i2, h):
        slot2 = jax.lax.rem(i2, NSLOTS)
        lo = max(h * ht, 1)
        return pltpu.make_async_copy(
            xs_ref.at[slot2, lo:(h + 1) * ht],
            out_hbm.at[i2, lo - 1:(h + 1) * ht - 1, 0, :],
            sem.at[slot2, h])

    @pl.when(i >= NSLOTS)
    def _():
        for h in range(NH):
            _chunk_copy(i - NSLOTS, h).wait()

    def dot(a, wk):
        return jax.lax.dot_general(
            a, wk, dimension_numbers=(((1,), (1,)), ((), ())),
            preferred_element_type=jnp.float32)

    w0, w1, w2, w3, w4 = (w_ref[k].astype(jnp.bfloat16) for k in range(K))
    row = jax.lax.broadcasted_iota(jnp.int32, (n, n), 0)
    col = jax.lax.broadcasted_iota(jnp.int32, (n, n), 1)
    w4 = jnp.where(row == col, 0.0, w4)
    bias = jnp.broadcast_to(b_ref[0][None, :], (ht, n)).astype(jnp.float32)
    wcomb = jnp.concatenate(
        [jnp.concatenate([w2, w4], axis=1),
         jnp.concatenate([w1, w3], axis=1)], axis=0)

    for h in range(NH):
        base = h * ht
        xch = xc_ref[0, base:base + ht].astype(jnp.bfloat16)
        if h == 0:
            halo8 = jnp.zeros((8, n), jnp.bfloat16)
            hrow = jnp.zeros((1, n), jnp.float32)
        else:
            halo8 = xc_ref[0, base - 8:base].astype(jnp.bfloat16)
            hrow = dot(halo8, w1)[5:6] + dot(halo8, w3)[7:8]
        s0 = jnp.concatenate([halo8[4:8], xch[:ht - 4]], axis=0)
        s2 = jnp.concatenate([halo8[6:8], xch[:ht - 2]], axis=0)

        a2 = jnp.concatenate([s2, xch], axis=1)
        y = dot(a2, wcomb)
        odd = y[:, n:]
        odd_sh = jnp.concatenate([hrow, odd[:ht - 1]], axis=0)
        acc = bias + dot(s0, w0) + y[:, :n] + odd_sh
        xs_ref[slot, base:base + ht] = acc
        _chunk_copy(i, h).start()

    @pl.when(i == nb - 1)
    def _():
        for d in range(NSLOTS - 1, -1, -1):
            @pl.when(nb >= d + 1)
            def _():
                for h in range(NH):
                    _chunk_copy(i - d, h).wait()


@functools.partial(jax.jit, static_argnames=())
def kernel(spikes, weight, bias):
    b, t, n, _ = spikes.shape
    x = jnp.reshape(spikes, (b, t, n))
    w = jnp.transpose(weight, (2, 0, 1))
    bias2 = bias[None, :]
    out = pl.pallas_call(
        _conv_body,
        grid=(b,),
        in_specs=[
            pl.BlockSpec((1, t, n), lambda i: (i, 0, 0)),
            pl.BlockSpec((K, n, n), lambda i: (0, 0, 0)),
            pl.BlockSpec((1, n), lambda i: (0, 0)),
        ],
        out_specs=pl.BlockSpec(memory_space=pl.ANY),
        out_shape=jax.ShapeDtypeStruct((b, t - 1, 1, n), jnp.float32),
        scratch_shapes=[
            pltpu.MemorySpace.VMEM((NSLOTS, t, n), jnp.float32),
            pltpu.SemaphoreType.DMA((NSLOTS, NH)),
        ],
    )(x, w, bias2)
    return jnp.reshape(out, (b, t - 1, n, 1))


# speedup vs baseline: 1.0293x; 1.0293x over previous
import functools

import jax
import jax.numpy as jnp
from jax.experimental import pallas as pl
from jax.experimental.pallas import tpu as pltpu

NUM_VARS = 128
K = 5
NSLOTS = 4
NH = 4


def _conv_body(xc_ref, w_ref, b_ref, out_hbm, xs_ref, sem):
    i = pl.program_id(0)
    nb = pl.num_programs(0)
    tt = xc_ref.shape[1]
    ht = tt // NH
    slot = jax.lax.rem(i, NSLOTS)
    n = NUM_VARS

    def _chunk_copy(i2, h):
        slot2 = jax.lax.rem(i2, NSLOTS)
        lo = max(h * ht, 1)
        return pltpu.make_async_copy(
            xs_ref.at[slot2, lo:(h + 1) * ht],
            out_hbm.at[i2, lo - 1:(h + 1) * ht - 1, 0, :],
            sem.at[slot2, h])

    @pl.when(i >= NSLOTS)
    def _():
        for h in range(NH):
            _chunk_copy(i - NSLOTS, h).wait()

    def dot(a, wk):
        return jax.lax.dot_general(
            a, wk, dimension_numbers=(((1,), (1,)), ((), ())),
            preferred_element_type=jnp.float32)

    w0, w1, w2, w3, w4 = (w_ref[k].astype(jnp.bfloat16) for k in range(K))
    row = jax.lax.broadcasted_iota(jnp.int32, (n, n), 0)
    col = jax.lax.broadcasted_iota(jnp.int32, (n, n), 1)
    w4 = jnp.where(row == col, 0.0, w4)
    bias = jnp.broadcast_to(b_ref[0][None, :], (ht, n)).astype(jnp.float32)
    wcomb = jnp.concatenate(
        [jnp.concatenate([w2, w4], axis=1),
         jnp.concatenate([w1, w3], axis=1)], axis=0)

    for h in range(NH):
        base = h * ht
        xch = xc_ref[0, base:base + ht].astype(jnp.bfloat16)
        if h == 0:
            halo8 = jnp.zeros((8, n), jnp.bfloat16)
            hrow = jnp.zeros((1, n), jnp.float32)
        else:
            halo8 = xc_ref[0, base - 8:base].astype(jnp.bfloat16)
            hrow = dot(halo8, w1)[5:6] + dot(halo8, w3)[7:8]
        s0 = jnp.concatenate([halo8[4:8], xch[:ht - 4]], axis=0)
        s2 = jnp.concatenate([halo8[6:8], xch[:ht - 2]], axis=0)

        a2 = jnp.concatenate([s2, xch], axis=1)
        y = dot(a2, wcomb)
        odd = y[:, n:]
        odd_sh = jnp.concatenate([hrow, odd[:ht - 1]], axis=0)
        acc = bias + dot(s0, w0) + y[:, :n] + odd_sh
        xs_ref[slot, base:base + ht] = acc
        _chunk_copy(i, h).start()

    @pl.when(i == nb - 1)
    def _():
        for d in range(NSLOTS - 1, -1, -1):
            @pl.when(nb >= d + 1)
            def _():
                for h in range(NH):
                    _chunk_copy(i - d, h).wait()


@functools.partial(jax.jit, static_argnames=())
def kernel(spikes, weight, bias):
    b, t, n, _ = spikes.shape
    x = jnp.reshape(spikes, (b, t, n))
    w = jnp.transpose(weight, (2, 0, 1))
    bias2 = bias[None, :]
    out = pl.pallas_call(
        _conv_body,
        grid=(b,),
        in_specs=[
            pl.BlockSpec((1, t, n), lambda i: (i, 0, 0)),
            pl.BlockSpec((K, n, n), lambda i: (0, 0, 0)),
            pl.BlockSpec((1, n), lambda i: (0, 0)),
        ],
        out_specs=pl.BlockSpec(memory_space=pl.ANY),
        out_shape=jax.ShapeDtypeStruct((b, t - 1, 1, n), jnp.float32),
        scratch_shapes=[
            pltpu.MemorySpace.VMEM((NSLOTS, t, n), jnp.float32),
            pltpu.SemaphoreType.DMA((NSLOTS, NH)),
        ],
    )(x, w, bias2)
    return jnp.reshape(out, (b, t - 1, n, 1))
